# Initial kernel scaffold; baseline (speedup 1.0000x reference)
import jax
import jax.numpy as jnp
from jax.experimental import pallas as pl


def kernel(node_feat, edge_attr, pos, Wn, bn, We, be, We1, be1, We2, be2, Wx1, bx1, Wx2, bx2, Wh1, bh1, Wh2, bh2, ln_g, ln_b, edge_index):
    raise NotImplementedError("write your pallas kernel here")


# speedup vs baseline: 2.4288x; 2.4288x over previous
import functools

import jax
import jax.numpy as jnp
from jax import lax
from jax.experimental import pallas as pl
from jax.experimental.pallas import tpu as pltpu
from jax.experimental.pallas import tpu_sc as plsc

_N = 10000
_E = 320000
_D = 128
_L = 3
_NRBF = 32
_RBF_MAX = 10.0
_RES_SCALE = 1000.0
_PW = 16

_NC = 2
_NS = 16
_NW = _NC * _NS
_EPW = _E // _NW
_CHUNK = 200
_NLOOP = _EPW // _CHUNK

_BE_EMB = 4000
_BE = 2000
_BN = 2000

_f32 = jnp.float32


def _silu(x):
    return x / (1.0 + jnp.exp(-x))


def _node_embed_body(nf_ref, w_ref, b_ref, o_ref):
    o_ref[...] = jnp.dot(nf_ref[...], w_ref[...],
                         preferred_element_type=_f32) + b_ref[...]


def _node_embed(nf8, Wn8, bn):
    return pl.pallas_call(
        _node_embed_body,
        out_shape=jax.ShapeDtypeStruct((_N, _D), _f32),
    )(nf8, Wn8, bn)


def _edge_embed_body(ea_ref, we_ref, be_ref, o_ref):
    d = ea_ref[...]
    cen = lax.broadcasted_iota(jnp.int32, (1, _NRBF), 1).astype(_f32) * (
        _RBF_MAX / (_NRBF - 1))
    gamma = 1.0 / ((_RBF_MAX / _NRBF) ** 2)
    r = jnp.exp(-gamma * (d - cen) ** 2)
    o_ref[...] = jnp.dot(r, we_ref[...], preferred_element_type=_f32) + be_ref[...]


def _edge_embed(edge_attr, We, be):
    nblk = _E // _BE_EMB
    return pl.pallas_call(
        _edge_embed_body,
        grid=(nblk,),
        in_specs=[
            pl.BlockSpec((_BE_EMB, 1), lambda i: (i, 0)),
            pl.BlockSpec((_NRBF, _D), lambda i: (0, 0)),
            pl.BlockSpec((1, _D), lambda i: (0, 0)),
        ],
        out_specs=pl.BlockSpec((_BE_EMB, _D), lambda i: (i, 0)),
        out_shape=jax.ShapeDtypeStruct((_E, _D), _f32),
    )(edge_attr, We, be)


def _edge_mlp_body(hs_ref, hd_ref, e_ref, ps_ref, pd_ref,
                   w1hd_ref, w1hs_ref, w1e_ref, w1d2_ref, b1_ref,
                   w2_ref, b2_ref, wx1_ref, bx1_ref, wx2_ref, bx2_ref,
                   m_ref, wrel_ref):
    rel = pd_ref[...] - ps_ref[...]
    d2 = jnp.sum(rel * rel, axis=1, keepdims=True)
    z = (jnp.dot(hd_ref[...], w1hd_ref[...], preferred_element_type=_f32)
         + jnp.dot(hs_ref[...], w1hs_ref[...], preferred_element_type=_f32)
         + jnp.dot(e_ref[...], w1e_ref[...], preferred_element_type=_f32)
         + d2 * w1d2_ref[...] + b1_ref[...])
    m1 = _silu(z)
    m = _silu(jnp.dot(m1, w2_ref[...], preferred_element_type=_f32) + b2_ref[...])
    t = _silu(jnp.dot(m, wx1_ref[...], preferred_element_type=_f32) + bx1_ref[...])
    w = jnp.sum(t * wx2_ref[...], axis=1, keepdims=True) + bx2_ref[...]
    m_ref[...] = m
    ones_col = (lax.broadcasted_iota(jnp.int32, (1, _PW), 1) == 3).astype(_f32)
    wrel_ref[...] = rel * w + ones_col


def _edge_mlp(hs, hd, e, ps, pd, w1hd, w1hs, w1e, w1d2, b1,
              w2, b2, wx1, bx1, wx2, bx2):
    nblk = _E // _BE
    row = lambda i: (i, 0)
    full = lambda i: (0, 0)
    return pl.pallas_call(
        _edge_mlp_body,
        grid=(nblk,),
        in_specs=[
            pl.BlockSpec((_BE, _D), row),
            pl.BlockSpec((_BE, _D), row),
            pl.BlockSpec((_BE, _D), row),
            pl.BlockSpec((_BE, _PW), row),
            pl.BlockSpec((_BE, _PW), row),
            pl.BlockSpec((_D, _D), full),
            pl.BlockSpec((_D, _D), full),
            pl.BlockSpec((_D, _D), full),
            pl.BlockSpec((1, _D), full),
            pl.BlockSpec((1, _D), full),
            pl.BlockSpec((_D, _D), full),
            pl.BlockSpec((1, _D), full),
            pl.BlockSpec((_D, _D), full),
            pl.BlockSpec((1, _D), full),
            pl.BlockSpec((1, _D), full),
            pl.BlockSpec((1, 1), full),
        ],
        out_specs=[
            pl.BlockSpec((_BE, _D), row),
            pl.BlockSpec((_BE, _PW), row),
        ],
        out_shape=[
            jax.ShapeDtypeStruct((_E, _D), _f32),
            jax.ShapeDtypeStruct((_E, _PW), _f32),
        ],
    )(hs, hd, e, ps, pd, w1hd, w1hs, w1e, w1d2, b1, w2, b2, wx1, bx1, wx2, bx2)


def _node_update_body(h_ref, p_ref, a0_ref, a1_ref, c0_ref, c1_ref,
                      wh1h_ref, wh1a_ref, bh1_ref, wh2_ref, bh2_ref,
                      g_ref, b_ref, ho_ref, po_ref):
    h = h_ref[...]
    agg = a0_ref[...] + a1_ref[...]
    crd = c0_ref[...] + c1_ref[...]
    deg = crd[:, 3:4]
    posmask = (lax.broadcasted_iota(jnp.int32, (1, _PW), 1) < 3).astype(_f32)
    po_ref[...] = p_ref[...] + crd * posmask / (deg + 1.0)
    u = _silu(jnp.dot(h, wh1h_ref[...], preferred_element_type=_f32)
              + jnp.dot(agg, wh1a_ref[...], preferred_element_type=_f32)
              + bh1_ref[...])
    h2 = h + jnp.dot(u, wh2_ref[...], preferred_element_type=_f32) + bh2_ref[...]
    mu = jnp.mean(h2, axis=1, keepdims=True)
    dc = h2 - mu
    var = jnp.mean(dc * dc, axis=1, keepdims=True)
    ho_ref[...] = dc * lax.rsqrt(var + 1e-5) * g_ref[...] + b_ref[...]


def _node_update(h, pos16, a0, a1, c0, c1, wh1h, wh1a, bh1, wh2, bh2, g, b):
    nblk = _N // _BN
    row = lambda i: (i, 0)
    full = lambda i: (0, 0)
    return pl.pallas_call(
        _node_update_body,
        grid=(nblk,),
        in_specs=[
            pl.BlockSpec((_BN, _D), row),
            pl.BlockSpec((_BN, _PW), row),
            pl.BlockSpec((_BN, _D), row),
            pl.BlockSpec((_BN, _D), row),
            pl.BlockSpec((_BN, _PW), row),
            pl.BlockSpec((_BN, _PW), row),
            pl.BlockSpec((_D, _D), full),
            pl.BlockSpec((_D, _D), full),
            pl.BlockSpec((1, _D), full),
            pl.BlockSpec((_D, _D), full),
            pl.BlockSpec((1, _D), full),
            pl.BlockSpec((1, _D), full),
            pl.BlockSpec((1, _D), full),
        ],
        out_specs=[
            pl.BlockSpec((_BN, _D), row),
            pl.BlockSpec((_BN, _PW), row),
        ],
        out_shape=[
            jax.ShapeDtypeStruct((_N, _D), _f32),
            jax.ShapeDtypeStruct((_N, _PW), _f32),
        ],
    )(h, pos16, a0, a1, c0, c1, wh1h, wh1a, bh1, wh2, bh2, g, b)


def _sc_gather(h, pos16, src, dst):
    mesh = plsc.VectorSubcoreMesh(core_axis_name="c", subcore_axis_name="s")

    @functools.partial(
        pl.kernel,
        mesh=mesh,
        out_type=[
            jax.ShapeDtypeStruct((_E, _D), _f32),
            jax.ShapeDtypeStruct((_E, _D), _f32),
            jax.ShapeDtypeStruct((_E, _PW), _f32),
            jax.ShapeDtypeStruct((_E, _PW), _f32),
        ],
        scratch_types=[
            pltpu.VMEM((_CHUNK,), jnp.int32),
            pltpu.VMEM((_CHUNK,), jnp.int32),
            pltpu.VMEM((_CHUNK, _D), _f32),
            pltpu.VMEM((_CHUNK, _D), _f32),
            pltpu.VMEM((_CHUNK, _PW), _f32),
            pltpu.VMEM((_CHUNK, _PW), _f32),
            pltpu.SemaphoreType.DMA,
        ],
        compiler_params=pltpu.CompilerParams(use_tc_tiling_on_sc=False),
    )
    def k(h_hbm, p_hbm, src_hbm, dst_hbm,
          hs_hbm, hd_hbm, ps_hbm, pd_hbm,
          isv, idv, bhs, bhd, bps, bpd, sem):
        c = lax.axis_index("c")
        s = lax.axis_index("s")
        base = (s * _NC + c) * _EPW

        def body(i, carry):
            off = pl.multiple_of(base + i * _CHUNK, 8)
            pltpu.sync_copy(src_hbm.at[pl.ds(off, _CHUNK)], isv)
            pltpu.sync_copy(dst_hbm.at[pl.ds(off, _CHUNK)], idv)
            pltpu.async_copy(h_hbm.at[isv], bhs, sem).wait()
            pltpu.async_copy(h_hbm.at[idv], bhd, sem).wait()
            pltpu.async_copy(p_hbm.at[isv], bps, sem).wait()
            pltpu.async_copy(p_hbm.at[idv], bpd, sem).wait()
            pltpu.sync_copy(bhs, hs_hbm.at[pl.ds(off, _CHUNK)])
            pltpu.sync_copy(bhd, hd_hbm.at[pl.ds(off, _CHUNK)])
            pltpu.sync_copy(bps, ps_hbm.at[pl.ds(off, _CHUNK)])
            pltpu.sync_copy(bpd, pd_hbm.at[pl.ds(off, _CHUNK)])
            return carry

        lax.fori_loop(0, _NLOOP, body, 0)

    return k(h, pos16, src, dst)


def _sc_scatter(m, wrel, dst, z128, z16):
    mesh = plsc.VectorSubcoreMesh(core_axis_name="c", subcore_axis_name="s")

    @functools.partial(
        pl.kernel,
        mesh=mesh,
        out_type=[
            jax.ShapeDtypeStruct((_NC, _N, _D), _f32),
            jax.ShapeDtypeStruct((_NC, _N, _PW), _f32),
        ],
        scratch_types=[
            pltpu.VMEM((_CHUNK,), jnp.int32),
            pltpu.VMEM((_CHUNK, _D), _f32),
            pltpu.VMEM((_CHUNK, _PW), _f32),
            pltpu.VMEM_SHARED((_N, _D), _f32),
            pltpu.VMEM_SHARED((_N, _PW), _f32),
        ],
        compiler_params=pltpu.CompilerParams(use_tc_tiling_on_sc=False),
    )
    def k(m_hbm, w_hbm, dst_hbm, z128_hbm, z16_hbm,
          agg_hbm, crd_hbm,
          idx, bufm, bufw, sh_agg, sh_crd):
        c = lax.axis_index("c")
        s = lax.axis_index("s")
        base = (c * _NS + s) * _EPW

        @pl.when(s == 0)
        def _():
            pltpu.sync_copy(z128_hbm, sh_agg)
            pltpu.sync_copy(z16_hbm, sh_crd)

        plsc.subcore_barrier()

        def body(i, carry):
            off = pl.multiple_of(base + i * _CHUNK, 8)
            pltpu.sync_copy(dst_hbm.at[pl.ds(off, _CHUNK)], idx)
            pltpu.sync_copy(m_hbm.at[pl.ds(off, _CHUNK)], bufm)
            pltpu.sync_copy(w_hbm.at[pl.ds(off, _CHUNK)], bufw)
            pltpu.sync_copy(bufm, sh_agg.at[idx], add=True)
            pltpu.sync_copy(bufw, sh_crd.at[idx], add=True)
            return carry

        lax.fori_loop(0, _NLOOP, body, 0)

        plsc.subcore_barrier()

        @pl.when(s == 0)
        def _():
            pltpu.sync_copy(sh_agg, agg_hbm.at[c])
            pltpu.sync_copy(sh_crd, crd_hbm.at[c])

    return k(m, wrel, dst, z128, z16)


def kernel(node_feat, edge_attr, pos, Wn, bn, We, be, We1, be1, We2, be2,
           Wx1, bx1, Wx2, bx2, Wh1, bh1, Wh2, bh2, ln_g, ln_b, edge_index):
    src = edge_index[0]
    dst = edge_index[1]

    nf8 = jnp.concatenate(
        [node_feat[:, :6], node_feat[:, 6:7] / _RES_SCALE,
         jnp.zeros((_N, 1), _f32)], axis=1)
    Wn8 = jnp.concatenate([Wn, jnp.zeros((1, _D), _f32)], axis=0)
    pos16 = jnp.concatenate([pos, jnp.zeros((_N, _PW - 3), _f32)], axis=1)

    z128 = jnp.zeros((_N, _D), _f32)
    z16 = jnp.zeros((_N, _PW), _f32)

    h = _node_embed(nf8, Wn8, bn.reshape(1, _D))
    e = _edge_embed(edge_attr, We, be.reshape(1, _D))

    for l in range(_L):
        w1hd = We1[l, 0:_D]
        w1hs = We1[l, _D:2 * _D]
        w1d2 = We1[l, 2 * _D:2 * _D + 1]
        w1e = We1[l, 2 * _D + 1:]

        hs, hd, ps, pd = _sc_gather(h, pos16, src, dst)
        m, wrel = _edge_mlp(hs, hd, e, ps, pd,
                            w1hd, w1hs, w1e, w1d2, be1[l].reshape(1, _D),
                            We2[l], be2[l].reshape(1, _D),
                            Wx1[l], bx1[l].reshape(1, _D),
                            Wx2[l].reshape(1, _D), bx2[l].reshape(1, 1))
        aggp, crdp = _sc_scatter(m, wrel, dst, z128, z16)
        h, pos16 = _node_update(h, pos16, aggp[0], aggp[1], crdp[0], crdp[1],
                                Wh1[l, :_D], Wh1[l, _D:],
                                bh1[l].reshape(1, _D),
                                Wh2[l], bh2[l].reshape(1, _D),
                                ln_g[l].reshape(1, _D), ln_b[l].reshape(1, _D))

    return h, pos16[:, :3]
